# baseline (device time: 42327 ns/iter reference)
import jax
import jax.numpy as jnp
from jax import lax
from jax.experimental import pallas as pl
from jax.experimental.pallas import tpu as pltpu

N_DEV = 4
N_SUB = 256


def kernel(x, w_mat, scale_x, scale_w):
    m_per, k = x.shape
    n = w_mat.shape[1]
    n_per = n // N_DEV
    spb = n_per // N_SUB
    n_subs = N_DEV * spb

    def body(x_hbm, w_hbm, sx_ref, sw_ref, out_hbm,
             x_vmem, w_buf, y_ref, sc_ref, recv_q, recv_sc, stage,
             x_sem, copy_sems, out_sems, send_sems, recv_sems,
             sc_send_sems, sc_recv_sems):
        my = lax.axis_index("i")

        kh = k // 2
        xcp0 = pltpu.make_async_copy(
            x_hbm.at[:, pl.ds(0, kh)], x_vmem.at[:, pl.ds(0, kh)], x_sem.at[0]
        )
        xcp0.start()
        xcp1 = pltpu.make_async_copy(
            x_hbm.at[:, pl.ds(kh, kh)], x_vmem.at[:, pl.ds(kh, kh)], x_sem.at[1]
        )
        xcp1.start()

        def tq(i):
            blk = i // spb
            if blk < N_DEV - 1:
                return lax.rem(my + 1 + blk, N_DEV)
            return my

        def start_copy(i):
            cp = pltpu.make_async_copy(
                w_hbm.at[:, pl.ds(tq(i) * n_per + (i % spb) * N_SUB, N_SUB)],
                w_buf.at[i % 2],
                copy_sems.at[i % 2],
            )
            cp.start()
            return cp

        cps = [None] * n_subs
        cps[0] = start_copy(0)

        barrier = pltpu.get_barrier_semaphore()
        for d in range(1, N_DEV):
            pl.semaphore_signal(
                barrier, inc=1,
                device_id=(lax.rem(my + d, N_DEV),),
                device_id_type=pl.DeviceIdType.MESH,
            )
        pl.semaphore_wait(barrier, N_DEV - 1)

        s = sx_ref[0] * sw_ref[0]
        xcp0.wait()
        xb0 = x_vmem[:, pl.ds(0, kh)].astype(jnp.float8_e5m2)
        xb1 = None

        sends = []
        out_cps = []
        for i in range(n_subs):
            if i + 1 < n_subs:
                cps[i + 1] = start_copy(i + 1)
            cps[i].wait()
            acc = jnp.dot(
                xb0,
                w_buf[i % 2, pl.ds(0, kh), :].astype(jnp.float8_e5m2),
                preferred_element_type=jnp.float32,
            )
            if xb1 is None:
                xcp1.wait()
                xb1 = x_vmem[:, pl.ds(kh, kh)].astype(jnp.float8_e5m2)
            acc = acc + jnp.dot(
                xb1,
                w_buf[i % 2, pl.ds(kh, kh), :].astype(jnp.float8_e5m2),
                preferred_element_type=jnp.float32,
            )
            yblk = jnp.maximum(acc * s, 0.0)
            blk, j = i // spb, i % spb
            if blk < N_DEV - 1:
                rowmax = jnp.max(yblk, axis=1, keepdims=True)
                sc = jnp.maximum(rowmax, 1e-30) * (1.0 / 127.0)
                qv = jnp.round(yblk * (1.0 / sc)).astype(jnp.int8)
                y_ref[blk, j, :, :] = qv
                sc_ref[blk, j, :] = sc[:, 0]
                q = lax.rem(my + 1 + blk, N_DEV)
                rdma = pltpu.make_async_remote_copy(
                    src_ref=y_ref.at[blk, j],
                    dst_ref=recv_q.at[my, j],
                    send_sem=send_sems.at[blk, j],
                    recv_sem=recv_sems.at[my, j],
                    device_id=(q,),
                    device_id_type=pl.DeviceIdType.MESH,
                )
                rdma.start()
                sends.append(rdma)
                rdma_sc = pltpu.make_async_remote_copy(
                    src_ref=sc_ref.at[blk, j],
                    dst_ref=recv_sc.at[my, j],
                    send_sem=sc_send_sems.at[blk, j],
                    recv_sem=sc_recv_sems.at[my, j],
                    device_id=(q,),
                    device_id_type=pl.DeviceIdType.MESH,
                )
                rdma_sc.start()
                sends.append(rdma_sc)
            else:
                stage[0, :, pl.ds(j * N_SUB, N_SUB)] = yblk
                if j == spb - 1:
                    ocp = pltpu.make_async_copy(
                        stage.at[0],
                        out_hbm.at[pl.ds(my * m_per, m_per), :],
                        out_sems.at[0],
                    )
                    ocp.start()
                    out_cps.append(ocp)

        for d in range(1, N_DEV):
            src = lax.rem(my + N_DEV - d, N_DEV)
            stg = 1 + (d - 1) % 2
            for j in range(spb):
                recv = pltpu.make_async_remote_copy(
                    src_ref=y_ref.at[0, 0],
                    dst_ref=recv_q.at[src, j],
                    send_sem=send_sems.at[0, j],
                    recv_sem=recv_sems.at[src, j],
                    device_id=(src,),
                    device_id_type=pl.DeviceIdType.MESH,
                )
                recv.wait_recv()
                recv_s = pltpu.make_async_remote_copy(
                    src_ref=sc_ref.at[0, j],
                    dst_ref=recv_sc.at[src, j],
                    send_sem=sc_send_sems.at[0, j],
                    recv_sem=sc_recv_sems.at[src, j],
                    device_id=(src,),
                    device_id_type=pl.DeviceIdType.MESH,
                )
                recv_s.wait_recv()
            if d >= 3:
                out_cps[1].wait()
            for j in range(spb):
                scv = recv_sc[src, j, :][:, None]
                stage[stg, :, pl.ds(j * N_SUB, N_SUB)] = (
                    recv_q[src, j, :, :].astype(jnp.float32) * scv
                )
            ocp = pltpu.make_async_copy(
                stage.at[stg],
                out_hbm.at[pl.ds(src * m_per, m_per), :],
                out_sems.at[stg],
            )
            ocp.start()
            out_cps.append(ocp)

        for idx in (0, 2, 3):
            out_cps[idx].wait()
        for rdma in sends:
            rdma.wait_send()

    return pl.pallas_call(
        body,
        out_shape=jax.ShapeDtypeStruct((N_DEV * m_per, n_per), jnp.float32),
        in_specs=[
            pl.BlockSpec(memory_space=pltpu.MemorySpace.HBM),
            pl.BlockSpec(memory_space=pltpu.MemorySpace.HBM),
            pl.BlockSpec(memory_space=pltpu.SMEM),
            pl.BlockSpec(memory_space=pltpu.SMEM),
        ],
        out_specs=pl.BlockSpec(memory_space=pltpu.MemorySpace.HBM),
        scratch_shapes=[
            pltpu.VMEM((m_per, k), jnp.float32),
            pltpu.VMEM((2, k, N_SUB), jnp.float32),
            pltpu.VMEM((N_DEV - 1, spb, m_per, N_SUB), jnp.int8),
            pltpu.VMEM((N_DEV - 1, spb, m_per), jnp.float32),
            pltpu.VMEM((N_DEV, spb, m_per, N_SUB), jnp.int8),
            pltpu.VMEM((N_DEV, spb, m_per), jnp.float32),
            pltpu.VMEM((3, m_per, n_per), jnp.float32),
            pltpu.SemaphoreType.DMA((2,)),
            pltpu.SemaphoreType.DMA((2,)),
            pltpu.SemaphoreType.DMA((3,)),
            pltpu.SemaphoreType.DMA((N_DEV - 1, 2)),
            pltpu.SemaphoreType.DMA((N_DEV, 2)),
            pltpu.SemaphoreType.DMA((N_DEV - 1, 2)),
            pltpu.SemaphoreType.DMA((N_DEV, 2)),
        ],
        compiler_params=pltpu.CompilerParams(
            collective_id=0,
            vmem_limit_bytes=100 * 1024 * 1024,
        ),
    )(x, w_mat, scale_x, scale_w)
